# baseline (speedup 1.0000x reference)
import functools

import jax
import jax.numpy as jnp
from jax import lax
from jax.experimental import pallas as pl
from jax.experimental.pallas import tpu as pltpu
from jax.experimental.pallas import tpu_sc as plsc

_B = 4
_N = 2048
_REG_RATIO = 0.1
_RB = 1024
_NR = _N // _RB
_NW = 32
_CHUNK = _B * _N // _NW
_GROUPS = _CHUNK // 16
_SPLIT = 1024
_SC_ROWS = _B * _N - _SPLIT
_APW = _SC_ROWS // _NW
_AGR = _APW // 16
_RCH = 16
_NCH = _APW // _RCH


def _att_body(att_hbm, out_hbm, va0, va1, ca_v, sem0, sem1):
    wid = lax.axis_index("s") * 2 + lax.axis_index("c")
    base = _SPLIT + wid * _APW
    lanes = lax.iota(jnp.int32, 16)
    lanesf = lanes.astype(jnp.float32)
    pltpu.async_copy(att_hbm.at[pl.ds(base, _RCH)], va0, sem0)

    def row_sum(va, k):
        def jbody(j, accs):
            a0, a1, a2, a3 = accs
            o = j * 128
            jf = o.astype(jnp.float32) + lanesf
            a0 = a0 + va[k, pl.ds(o, 16)] * jf
            a1 = a1 + va[k, pl.ds(o + 16, 16)] * (jf + 16.0)
            a2 = a2 + va[k, pl.ds(o + 32, 16)] * (jf + 32.0)
            a3 = a3 + va[k, pl.ds(o + 48, 16)] * (jf + 48.0)
            a0 = a0 + va[k, pl.ds(o + 64, 16)] * (jf + 64.0)
            a1 = a1 + va[k, pl.ds(o + 80, 16)] * (jf + 80.0)
            a2 = a2 + va[k, pl.ds(o + 96, 16)] * (jf + 96.0)
            a3 = a3 + va[k, pl.ds(o + 112, 16)] * (jf + 112.0)
            return (a0, a1, a2, a3)
        z = jnp.zeros((16,), jnp.float32)
        a0, a1, a2, a3 = lax.fori_loop(0, _N // 128, jbody, (z, z, z, z))
        return jnp.sum((a0 + a1) + (a2 + a3))

    def chunk_body(c, ca_vec):
        even = (c & 1) == 0
        nxt = c + 1

        @pl.when(even)
        def _w0():
            pltpu.make_async_copy(att_hbm.at[pl.ds(0, _RCH)], va0, sem0).wait()

        @pl.when(~even)
        def _w1():
            pltpu.make_async_copy(att_hbm.at[pl.ds(0, _RCH)], va1, sem1).wait()

        @pl.when(nxt < _NCH)
        def _issue():
            off = base + nxt * _RCH

            @pl.when(even)
            def _i1():
                pltpu.async_copy(att_hbm.at[pl.ds(off, _RCH)], va1, sem1)

            @pl.when(~even)
            def _i0():
                pltpu.async_copy(att_hbm.at[pl.ds(off, _RCH)], va0, sem0)

        ca_vec = jnp.zeros((16,), jnp.float32)
        for k in range(_RCH):
            s0 = lax.cond(even,
                          lambda: row_sum(va0, k),
                          lambda: row_sum(va1, k))
            ca_vec = jnp.where(lanes == k, s0, ca_vec)

        ca_v[pl.ds(c * 16, 16)] = ca_vec.astype(jnp.int32)
        return 0

    lax.fori_loop(0, _NCH, chunk_body, 0)
    pltpu.sync_copy(ca_v, out_hbm.at[pl.ds(wid * _APW, _APW)])


def _make_att_kernel():
    return functools.partial(
        pl.kernel,
        mesh=plsc.VectorSubcoreMesh(core_axis_name="c", subcore_axis_name="s"),
        out_type=jax.ShapeDtypeStruct((_SC_ROWS,), jnp.int32),
        compiler_params=pltpu.CompilerParams(needs_layout_passes=False),
        scratch_types=[
            pltpu.VMEM((_RCH, _N), jnp.float32),
            pltpu.VMEM((_RCH, _N), jnp.float32),
            pltpu.VMEM((_APW,), jnp.int32),
            pltpu.SemaphoreType.DMA,
            pltpu.SemaphoreType.DMA,
        ],
    )(_att_body)


def _gt_body(gt_ref, cg_ref, cgt_ref):
    ir = pl.program_id(1)
    G = gt_ref[0]
    col_iota = lax.broadcasted_iota(jnp.int32, (_N, 1), 0).astype(jnp.float32)
    row_iota = (lax.broadcasted_iota(jnp.int32, (1, _RB), 1) + ir * _RB).astype(jnp.float32)
    cg_ref[0, 0, :] = jnp.dot(G, col_iota,
                              preferred_element_type=jnp.float32)[:, 0].astype(jnp.int32)
    cgt_part = jnp.dot(row_iota, G, preferred_element_type=jnp.float32)[0]

    @pl.when(ir == 0)
    def _init():
        cgt_ref[0, 0, :] = cgt_part

    @pl.when(ir != 0)
    def _acc():
        cgt_ref[0, 0, :] = cgt_ref[0, 0, :] + cgt_part


def _gt_pass(gt):
    return pl.pallas_call(
        _gt_body,
        grid=(_B, _NR),
        in_specs=[pl.BlockSpec((1, _RB, _N), lambda b, ir: (b, ir, 0))],
        out_specs=[
            pl.BlockSpec((1, 1, _RB), lambda b, ir: (b * _NR + ir, 0, 0)),
            pl.BlockSpec((1, 1, _N), lambda b, ir: (b, 0, 0)),
        ],
        out_shape=[
            jax.ShapeDtypeStruct((_B * _NR, 1, _RB), jnp.int32),
            jax.ShapeDtypeStruct((_B, 1, _N), jnp.float32),
        ],
        compiler_params=pltpu.CompilerParams(
            dimension_semantics=("arbitrary", "arbitrary")),
    )(gt)


def _att0_body(att_ref, ca_ref):
    A = att_ref[0]
    col_iota = lax.broadcasted_iota(jnp.int32, (_N, 1), 0).astype(jnp.float32)
    ca_ref[0, 0, :] = jnp.dot(A, col_iota,
                              preferred_element_type=jnp.float32)[:, 0].astype(jnp.int32)


def _att0_pass(att):
    nb = _SPLIT // _RB
    return pl.pallas_call(
        _att0_body,
        grid=(nb,),
        in_specs=[pl.BlockSpec((1, _RB, _N), lambda i: (0, i, 0))],
        out_specs=[pl.BlockSpec((1, 1, _RB), lambda i: (i, 0, 0))],
        out_shape=[jax.ShapeDtypeStruct((nb, 1, _RB), jnp.int32)],
        compiler_params=pltpu.CompilerParams(
            dimension_semantics=("arbitrary",)),
    )(att)[0]


def _chase_body(cg_hbm, ca0_hbm, casc_hbm, cgt_hbm, src_hbm, sc_out, fl_out,
                ca_v, cgt_v, cg_v, src_v, scv, flv):
    wid = lax.axis_index("s") * 2 + lax.axis_index("c")
    base = wid * _CHUNK
    b = base // _N
    ibase = base - b * _N

    @pl.when(base < _SPLIT)
    def _head():
        pltpu.sync_copy(ca0_hbm.at[pl.ds(base, _CHUNK)], ca_v)

    @pl.when(base >= _SPLIT)
    def _tail():
        off = pl.multiple_of(jnp.maximum(base - _SPLIT, 0), 8)
        pltpu.sync_copy(casc_hbm.at[pl.ds(off, _CHUNK)], ca_v)

    pltpu.sync_copy(cgt_hbm.at[pl.ds(b * _N, _N)], cgt_v)
    pltpu.sync_copy(cg_hbm.at[pl.ds(b * _N, _N)], cg_v)
    pltpu.sync_copy(src_hbm, src_v)
    s_vec = plsc.load_gather(src_v, [jnp.full((16,), b, jnp.int32)])
    lanes = lax.iota(jnp.int32, 16)
    for g in range(_GROUPS):
        i16 = ibase + g * 16 + lanes
        a16 = ca_v[pl.ds(g * 16, 16)]
        m16 = plsc.load_gather(cgt_v, [a16]).astype(jnp.int32)
        sc16 = plsc.load_gather(cg_v, [m16])
        flag = (i16 < s_vec) & (m16 != i16)
        scv[pl.ds(g * 16, 16)] = sc16
        flv[pl.ds(g * 16, 16)] = flag.astype(jnp.int32)
    pltpu.sync_copy(scv, sc_out.at[pl.ds(base, _CHUNK)])
    pltpu.sync_copy(flv, fl_out.at[pl.ds(base, _CHUNK)])


def _make_chase_kernel():
    return functools.partial(
        pl.kernel,
        mesh=plsc.VectorSubcoreMesh(core_axis_name="c", subcore_axis_name="s"),
        out_type=[
            jax.ShapeDtypeStruct((_B * _N,), jnp.int32),
            jax.ShapeDtypeStruct((_B * _N,), jnp.int32),
        ],
        compiler_params=pltpu.CompilerParams(needs_layout_passes=False),
        scratch_types=[
            pltpu.VMEM((_CHUNK,), jnp.int32),
            pltpu.VMEM((_N,), jnp.float32),
            pltpu.VMEM((_N,), jnp.int32),
            pltpu.VMEM((16,), jnp.int32),
            pltpu.VMEM((_CHUNK,), jnp.int32),
            pltpu.VMEM((_CHUNK,), jnp.int32),
        ],
    )(_chase_body)


def _pred_body(src_ref, tgt_ref, pred_ref, cg_ref, sc_ref, fl_ref,
               out_ref, acc_ref):
    b = pl.program_id(0)
    ir = pl.program_id(1)
    s = src_ref[b]
    t = tgt_ref[b]
    P = pred_ref[0]
    rows = lax.broadcasted_iota(jnp.int32, (_RB, _N), 0) + ir * _RB
    cols = lax.broadcasted_iota(jnp.int32, (_RB, _N), 1)
    rv = rows < s
    region = rv & (cols < t)
    l1mp = jnp.maximum(jnp.log(1.0 - P), -100.0)
    cgv = cg_ref[0, 0, :][:, None]
    scv = sc_ref[0, 0, :][:, None]
    ones_n = jnp.ones((_N, 1), jnp.float32)
    terms = jnp.where(region, l1mp, 0.0)
    s1 = -jnp.sum(jnp.dot(terms, ones_n, preferred_element_type=jnp.float32))
    p1 = jnp.dot(jnp.where(cols == cgv, P, 0.0), ones_n,
                 preferred_element_type=jnp.float32)[:, 0]
    p2 = jnp.dot(jnp.where(cols == scv, P, 0.0), ones_n,
                 preferred_element_type=jnp.float32)[:, 0]
    lp1 = jnp.maximum(jnp.log(p1), -100.0)
    l1mp1 = jnp.maximum(jnp.log(1.0 - p1), -100.0)
    corr = jnp.sum(jnp.where(rv[:, 0], l1mp1 - lp1, 0.0))
    flg = fl_ref[0, 0, :] != 0
    reg = jnp.sum(jnp.where(flg, p1 - p2, 0.0))
    part = s1 + corr - _REG_RATIO * reg

    @pl.when((b == 0) & (ir == 0))
    def _first():
        acc_ref[0] = part

    @pl.when((b != 0) | (ir != 0))
    def _rest():
        acc_ref[0] = acc_ref[0] + part

    @pl.when((b == _B - 1) & (ir == _NR - 1))
    def _flush():
        nsum = (src_ref[0] + src_ref[1] + src_ref[2] + src_ref[3]).astype(jnp.float32)
        out_ref[0] = acc_ref[0] / nsum


def _pred_pass(pred, cg, sc, fl, src_i, tgt_i):
    rowspec = pl.BlockSpec((1, 1, _RB), lambda b, ir: (b * _NR + ir, 0, 0))
    return pl.pallas_call(
        _pred_body,
        grid=(_B, _NR),
        in_specs=[
            pl.BlockSpec(memory_space=pltpu.SMEM),
            pl.BlockSpec(memory_space=pltpu.SMEM),
            pl.BlockSpec((1, _RB, _N), lambda b, ir: (b, ir, 0)),
            rowspec,
            rowspec,
            rowspec,
        ],
        out_specs=[pl.BlockSpec(memory_space=pltpu.SMEM)],
        out_shape=[jax.ShapeDtypeStruct((1,), jnp.float32)],
        scratch_shapes=[pltpu.SMEM((1,), jnp.float32)],
        compiler_params=pltpu.CompilerParams(
            dimension_semantics=("arbitrary", "arbitrary")),
    )(src_i, tgt_i, pred, cg, sc, fl)


def kernel(pred_dsmat, pred_perm, pred_perm_att, gt_perm, src_ns, tgt_ns):
    pred = pred_dsmat.astype(jnp.float32)
    gt = gt_perm.astype(jnp.float32)
    att = pred_perm_att.astype(jnp.float32)
    src_i = src_ns.astype(jnp.int32)
    tgt_i = tgt_ns.astype(jnp.int32)
    ca_sc = _make_att_kernel()(att.reshape(_B * _N, _N))
    cg, cgt = _gt_pass(gt)
    ca0 = _att0_pass(att)
    src_pad = jnp.zeros((16,), jnp.int32).at[:_B].set(src_i)
    sc, fl = _make_chase_kernel()(
        cg.reshape(_B * _N),
        ca0.reshape(_SPLIT),
        ca_sc,
        cgt.reshape(_B * _N),
        src_pad,
    )
    out = _pred_pass(
        pred,
        cg,
        sc.reshape(_B * _NR, 1, _RB),
        fl.reshape(_B * _NR, 1, _RB),
        src_i,
        tgt_i,
    )[0]
    return out[0]


# speedup vs baseline: 1.0231x; 1.0231x over previous
import functools

import jax
import jax.numpy as jnp
from jax import lax
from jax.experimental import pallas as pl
from jax.experimental.pallas import tpu as pltpu
from jax.experimental.pallas import tpu_sc as plsc

_B = 4
_N = 2048
_REG_RATIO = 0.1
_RB = 1024
_NR = _N // _RB
_NW = 32
_CHUNK = _B * _N // _NW
_GROUPS = _CHUNK // 16
_SPLIT = 2048
_SC_ROWS = _B * _N - _SPLIT
_APW = _SC_ROWS // _NW
_AGR = _APW // 16
_RCH = 16
_NCH = _APW // _RCH


def _att_body(att_hbm, out_hbm, va0, va1, ca_v, sem0, sem1):
    wid = lax.axis_index("s") * 2 + lax.axis_index("c")
    base = _SPLIT + wid * _APW
    lanes = lax.iota(jnp.int32, 16)
    lanesf = lanes.astype(jnp.float32)
    pltpu.async_copy(att_hbm.at[pl.ds(base, _RCH)], va0, sem0)

    def row_sum(va, k):
        def jbody(j, accs):
            a0, a1, a2, a3 = accs
            o = j * 128
            jf = o.astype(jnp.float32) + lanesf
            a0 = a0 + va[k, pl.ds(o, 16)] * jf
            a1 = a1 + va[k, pl.ds(o + 16, 16)] * (jf + 16.0)
            a2 = a2 + va[k, pl.ds(o + 32, 16)] * (jf + 32.0)
            a3 = a3 + va[k, pl.ds(o + 48, 16)] * (jf + 48.0)
            a0 = a0 + va[k, pl.ds(o + 64, 16)] * (jf + 64.0)
            a1 = a1 + va[k, pl.ds(o + 80, 16)] * (jf + 80.0)
            a2 = a2 + va[k, pl.ds(o + 96, 16)] * (jf + 96.0)
            a3 = a3 + va[k, pl.ds(o + 112, 16)] * (jf + 112.0)
            return (a0, a1, a2, a3)
        z = jnp.zeros((16,), jnp.float32)
        a0, a1, a2, a3 = lax.fori_loop(0, _N // 128, jbody, (z, z, z, z))
        return jnp.sum((a0 + a1) + (a2 + a3))

    def chunk_body(c, ca_vec):
        even = (c & 1) == 0
        nxt = c + 1

        @pl.when(even)
        def _w0():
            pltpu.make_async_copy(att_hbm.at[pl.ds(0, _RCH)], va0, sem0).wait()

        @pl.when(~even)
        def _w1():
            pltpu.make_async_copy(att_hbm.at[pl.ds(0, _RCH)], va1, sem1).wait()

        @pl.when(nxt < _NCH)
        def _issue():
            off = base + nxt * _RCH

            @pl.when(even)
            def _i1():
                pltpu.async_copy(att_hbm.at[pl.ds(off, _RCH)], va1, sem1)

            @pl.when(~even)
            def _i0():
                pltpu.async_copy(att_hbm.at[pl.ds(off, _RCH)], va0, sem0)

        ca_vec = jnp.zeros((16,), jnp.float32)
        for k in range(_RCH):
            s0 = lax.cond(even,
                          lambda: row_sum(va0, k),
                          lambda: row_sum(va1, k))
            ca_vec = jnp.where(lanes == k, s0, ca_vec)

        ca_v[pl.ds(c * 16, 16)] = ca_vec.astype(jnp.int32)
        return 0

    lax.fori_loop(0, _NCH, chunk_body, 0)
    pltpu.sync_copy(ca_v, out_hbm.at[pl.ds(wid * _APW, _APW)])


def _make_att_kernel():
    return functools.partial(
        pl.kernel,
        mesh=plsc.VectorSubcoreMesh(core_axis_name="c", subcore_axis_name="s"),
        out_type=jax.ShapeDtypeStruct((_SC_ROWS,), jnp.int32),
        compiler_params=pltpu.CompilerParams(needs_layout_passes=False),
        scratch_types=[
            pltpu.VMEM((_RCH, _N), jnp.float32),
            pltpu.VMEM((_RCH, _N), jnp.float32),
            pltpu.VMEM((_APW,), jnp.int32),
            pltpu.SemaphoreType.DMA,
            pltpu.SemaphoreType.DMA,
        ],
    )(_att_body)


def _gt_body(gt_ref, cg_ref, cgt_ref):
    ir = pl.program_id(1)
    G = gt_ref[0]
    col_iota = lax.broadcasted_iota(jnp.int32, (_N, 1), 0).astype(jnp.float32)
    row_iota = (lax.broadcasted_iota(jnp.int32, (1, _RB), 1) + ir * _RB).astype(jnp.float32)
    cg_ref[0, 0, :] = jnp.dot(G, col_iota,
                              preferred_element_type=jnp.float32)[:, 0].astype(jnp.int32)
    cgt_part = jnp.dot(row_iota, G, preferred_element_type=jnp.float32)[0]

    @pl.when(ir == 0)
    def _init():
        cgt_ref[0, 0, :] = cgt_part

    @pl.when(ir != 0)
    def _acc():
        cgt_ref[0, 0, :] = cgt_ref[0, 0, :] + cgt_part


def _gt_pass(gt):
    return pl.pallas_call(
        _gt_body,
        grid=(_B, _NR),
        in_specs=[pl.BlockSpec((1, _RB, _N), lambda b, ir: (b, ir, 0))],
        out_specs=[
            pl.BlockSpec((1, 1, _RB), lambda b, ir: (b * _NR + ir, 0, 0)),
            pl.BlockSpec((1, 1, _N), lambda b, ir: (b, 0, 0)),
        ],
        out_shape=[
            jax.ShapeDtypeStruct((_B * _NR, 1, _RB), jnp.int32),
            jax.ShapeDtypeStruct((_B, 1, _N), jnp.float32),
        ],
        compiler_params=pltpu.CompilerParams(
            dimension_semantics=("arbitrary", "arbitrary")),
    )(gt)


def _att0_body(att_ref, ca_ref):
    A = att_ref[0]
    col_iota = lax.broadcasted_iota(jnp.int32, (_N, 1), 0).astype(jnp.float32)
    ca_ref[0, 0, :] = jnp.dot(A, col_iota,
                              preferred_element_type=jnp.float32)[:, 0].astype(jnp.int32)


def _att0_pass(att):
    nb = _SPLIT // _RB
    return pl.pallas_call(
        _att0_body,
        grid=(nb,),
        in_specs=[pl.BlockSpec((1, _RB, _N), lambda i: (0, i, 0))],
        out_specs=[pl.BlockSpec((1, 1, _RB), lambda i: (i, 0, 0))],
        out_shape=[jax.ShapeDtypeStruct((nb, 1, _RB), jnp.int32)],
        compiler_params=pltpu.CompilerParams(
            dimension_semantics=("arbitrary",)),
    )(att)[0]


def _chase_body(cg_hbm, ca0_hbm, casc_hbm, cgt_hbm, src_hbm, sc_out, fl_out,
                ca_v, cgt_v, cg_v, src_v, scv, flv):
    wid = lax.axis_index("s") * 2 + lax.axis_index("c")
    base = wid * _CHUNK
    b = base // _N
    ibase = base - b * _N

    @pl.when(base < _SPLIT)
    def _head():
        pltpu.sync_copy(ca0_hbm.at[pl.ds(base, _CHUNK)], ca_v)

    @pl.when(base >= _SPLIT)
    def _tail():
        off = pl.multiple_of(jnp.maximum(base - _SPLIT, 0), 8)
        pltpu.sync_copy(casc_hbm.at[pl.ds(off, _CHUNK)], ca_v)

    pltpu.sync_copy(cgt_hbm.at[pl.ds(b * _N, _N)], cgt_v)
    pltpu.sync_copy(cg_hbm.at[pl.ds(b * _N, _N)], cg_v)
    pltpu.sync_copy(src_hbm, src_v)
    s_vec = plsc.load_gather(src_v, [jnp.full((16,), b, jnp.int32)])
    lanes = lax.iota(jnp.int32, 16)
    for g in range(_GROUPS):
        i16 = ibase + g * 16 + lanes
        a16 = ca_v[pl.ds(g * 16, 16)]
        m16 = plsc.load_gather(cgt_v, [a16]).astype(jnp.int32)
        sc16 = plsc.load_gather(cg_v, [m16])
        flag = (i16 < s_vec) & (m16 != i16)
        scv[pl.ds(g * 16, 16)] = sc16
        flv[pl.ds(g * 16, 16)] = flag.astype(jnp.int32)
    pltpu.sync_copy(scv, sc_out.at[pl.ds(base, _CHUNK)])
    pltpu.sync_copy(flv, fl_out.at[pl.ds(base, _CHUNK)])


def _make_chase_kernel():
    return functools.partial(
        pl.kernel,
        mesh=plsc.VectorSubcoreMesh(core_axis_name="c", subcore_axis_name="s"),
        out_type=[
            jax.ShapeDtypeStruct((_B * _N,), jnp.int32),
            jax.ShapeDtypeStruct((_B * _N,), jnp.int32),
        ],
        compiler_params=pltpu.CompilerParams(needs_layout_passes=False),
        scratch_types=[
            pltpu.VMEM((_CHUNK,), jnp.int32),
            pltpu.VMEM((_N,), jnp.float32),
            pltpu.VMEM((_N,), jnp.int32),
            pltpu.VMEM((16,), jnp.int32),
            pltpu.VMEM((_CHUNK,), jnp.int32),
            pltpu.VMEM((_CHUNK,), jnp.int32),
        ],
    )(_chase_body)


def _pred_body(src_ref, tgt_ref, pred_ref, cg_ref, sc_ref, fl_ref,
               out_ref, acc_ref):
    b = pl.program_id(0)
    ir = pl.program_id(1)
    s = src_ref[b]
    t = tgt_ref[b]
    P = pred_ref[0]
    rows = lax.broadcasted_iota(jnp.int32, (_RB, _N), 0) + ir * _RB
    cols = lax.broadcasted_iota(jnp.int32, (_RB, _N), 1)
    rv = rows < s
    region = rv & (cols < t)
    l1mp = jnp.maximum(jnp.log(1.0 - P), -100.0)
    cgv = cg_ref[0, 0, :][:, None]
    scv = sc_ref[0, 0, :][:, None]
    ones_n = jnp.ones((_N, 1), jnp.float32)
    terms = jnp.where(region, l1mp, 0.0)
    s1 = -jnp.sum(jnp.dot(terms, ones_n, preferred_element_type=jnp.float32))
    p1 = jnp.dot(jnp.where(cols == cgv, P, 0.0), ones_n,
                 preferred_element_type=jnp.float32)[:, 0]
    p2 = jnp.dot(jnp.where(cols == scv, P, 0.0), ones_n,
                 preferred_element_type=jnp.float32)[:, 0]
    lp1 = jnp.maximum(jnp.log(p1), -100.0)
    l1mp1 = jnp.maximum(jnp.log(1.0 - p1), -100.0)
    corr = jnp.sum(jnp.where(rv[:, 0], l1mp1 - lp1, 0.0))
    flg = fl_ref[0, 0, :] != 0
    reg = jnp.sum(jnp.where(flg, p1 - p2, 0.0))
    part = s1 + corr - _REG_RATIO * reg

    @pl.when((b == 0) & (ir == 0))
    def _first():
        acc_ref[0] = part

    @pl.when((b != 0) | (ir != 0))
    def _rest():
        acc_ref[0] = acc_ref[0] + part

    @pl.when((b == _B - 1) & (ir == _NR - 1))
    def _flush():
        nsum = (src_ref[0] + src_ref[1] + src_ref[2] + src_ref[3]).astype(jnp.float32)
        out_ref[0] = acc_ref[0] / nsum


def _pred_pass(pred, cg, sc, fl, src_i, tgt_i):
    rowspec = pl.BlockSpec((1, 1, _RB), lambda b, ir: (b * _NR + ir, 0, 0))
    return pl.pallas_call(
        _pred_body,
        grid=(_B, _NR),
        in_specs=[
            pl.BlockSpec(memory_space=pltpu.SMEM),
            pl.BlockSpec(memory_space=pltpu.SMEM),
            pl.BlockSpec((1, _RB, _N), lambda b, ir: (b, ir, 0)),
            rowspec,
            rowspec,
            rowspec,
        ],
        out_specs=[pl.BlockSpec(memory_space=pltpu.SMEM)],
        out_shape=[jax.ShapeDtypeStruct((1,), jnp.float32)],
        scratch_shapes=[pltpu.SMEM((1,), jnp.float32)],
        compiler_params=pltpu.CompilerParams(
            dimension_semantics=("arbitrary", "arbitrary")),
    )(src_i, tgt_i, pred, cg, sc, fl)


def kernel(pred_dsmat, pred_perm, pred_perm_att, gt_perm, src_ns, tgt_ns):
    pred = pred_dsmat.astype(jnp.float32)
    gt = gt_perm.astype(jnp.float32)
    att = pred_perm_att.astype(jnp.float32)
    src_i = src_ns.astype(jnp.int32)
    tgt_i = tgt_ns.astype(jnp.int32)
    ca_sc = _make_att_kernel()(att.reshape(_B * _N, _N))
    cg, cgt = _gt_pass(gt)
    ca0 = _att0_pass(att)
    src_pad = jnp.zeros((16,), jnp.int32).at[:_B].set(src_i)
    sc, fl = _make_chase_kernel()(
        cg.reshape(_B * _N),
        ca0.reshape(_SPLIT),
        ca_sc,
        cgt.reshape(_B * _N),
        src_pad,
    )
    out = _pred_pass(
        pred,
        cg,
        sc.reshape(_B * _NR, 1, _RB),
        fl.reshape(_B * _NR, 1, _RB),
        src_i,
        tgt_i,
    )[0]
    return out[0]


# speedup vs baseline: 1.0234x; 1.0002x over previous
import functools

import jax
import jax.numpy as jnp
from jax import lax
from jax.experimental import pallas as pl
from jax.experimental.pallas import tpu as pltpu
from jax.experimental.pallas import tpu_sc as plsc

_B = 4
_N = 2048
_REG_RATIO = 0.1
_RB = 1024
_NR = _N // _RB
_NW = 32
_CHUNK = _B * _N // _NW
_GROUPS = _CHUNK // 16
_SPLIT = 3072
_SC_ROWS = _B * _N - _SPLIT
_APW = _SC_ROWS // _NW
_AGR = _APW // 16
_RCH = 16
_NCH = _APW // _RCH


def _att_body(att_hbm, out_hbm, va0, va1, ca_v, sem0, sem1):
    wid = lax.axis_index("s") * 2 + lax.axis_index("c")
    base = _SPLIT + wid * _APW
    lanes = lax.iota(jnp.int32, 16)
    lanesf = lanes.astype(jnp.float32)
    pltpu.async_copy(att_hbm.at[pl.ds(base, _RCH)], va0, sem0)

    def row_sum(va, k):
        def jbody(j, accs):
            a0, a1, a2, a3 = accs
            o = j * 128
            jf = o.astype(jnp.float32) + lanesf
            a0 = a0 + va[k, pl.ds(o, 16)] * jf
            a1 = a1 + va[k, pl.ds(o + 16, 16)] * (jf + 16.0)
            a2 = a2 + va[k, pl.ds(o + 32, 16)] * (jf + 32.0)
            a3 = a3 + va[k, pl.ds(o + 48, 16)] * (jf + 48.0)
            a0 = a0 + va[k, pl.ds(o + 64, 16)] * (jf + 64.0)
            a1 = a1 + va[k, pl.ds(o + 80, 16)] * (jf + 80.0)
            a2 = a2 + va[k, pl.ds(o + 96, 16)] * (jf + 96.0)
            a3 = a3 + va[k, pl.ds(o + 112, 16)] * (jf + 112.0)
            return (a0, a1, a2, a3)
        z = jnp.zeros((16,), jnp.float32)
        a0, a1, a2, a3 = lax.fori_loop(0, _N // 128, jbody, (z, z, z, z))
        return jnp.sum((a0 + a1) + (a2 + a3))

    def chunk_body(c, ca_vec):
        even = (c & 1) == 0
        nxt = c + 1

        @pl.when(even)
        def _w0():
            pltpu.make_async_copy(att_hbm.at[pl.ds(0, _RCH)], va0, sem0).wait()

        @pl.when(~even)
        def _w1():
            pltpu.make_async_copy(att_hbm.at[pl.ds(0, _RCH)], va1, sem1).wait()

        @pl.when(nxt < _NCH)
        def _issue():
            off = base + nxt * _RCH

            @pl.when(even)
            def _i1():
                pltpu.async_copy(att_hbm.at[pl.ds(off, _RCH)], va1, sem1)

            @pl.when(~even)
            def _i0():
                pltpu.async_copy(att_hbm.at[pl.ds(off, _RCH)], va0, sem0)

        ca_vec = jnp.zeros((16,), jnp.float32)
        for k in range(_RCH):
            s0 = lax.cond(even,
                          lambda: row_sum(va0, k),
                          lambda: row_sum(va1, k))
            ca_vec = jnp.where(lanes == k, s0, ca_vec)

        ca_v[pl.ds(c * 16, 16)] = ca_vec.astype(jnp.int32)
        return 0

    lax.fori_loop(0, _NCH, chunk_body, 0)
    pltpu.sync_copy(ca_v, out_hbm.at[pl.ds(wid * _APW, _APW)])


def _make_att_kernel():
    return functools.partial(
        pl.kernel,
        mesh=plsc.VectorSubcoreMesh(core_axis_name="c", subcore_axis_name="s"),
        out_type=jax.ShapeDtypeStruct((_SC_ROWS,), jnp.int32),
        compiler_params=pltpu.CompilerParams(needs_layout_passes=False),
        scratch_types=[
            pltpu.VMEM((_RCH, _N), jnp.float32),
            pltpu.VMEM((_RCH, _N), jnp.float32),
            pltpu.VMEM((_APW,), jnp.int32),
            pltpu.SemaphoreType.DMA,
            pltpu.SemaphoreType.DMA,
        ],
    )(_att_body)


def _gt_body(gt_ref, cg_ref, cgt_ref):
    ir = pl.program_id(1)
    G = gt_ref[0]
    col_iota = lax.broadcasted_iota(jnp.int32, (_N, 1), 0).astype(jnp.float32)
    row_iota = (lax.broadcasted_iota(jnp.int32, (1, _RB), 1) + ir * _RB).astype(jnp.float32)
    cg_ref[0, 0, :] = jnp.dot(G, col_iota,
                              preferred_element_type=jnp.float32)[:, 0].astype(jnp.int32)
    cgt_part = jnp.dot(row_iota, G, preferred_element_type=jnp.float32)[0]

    @pl.when(ir == 0)
    def _init():
        cgt_ref[0, 0, :] = cgt_part

    @pl.when(ir != 0)
    def _acc():
        cgt_ref[0, 0, :] = cgt_ref[0, 0, :] + cgt_part


def _gt_pass(gt):
    return pl.pallas_call(
        _gt_body,
        grid=(_B, _NR),
        in_specs=[pl.BlockSpec((1, _RB, _N), lambda b, ir: (b, ir, 0))],
        out_specs=[
            pl.BlockSpec((1, 1, _RB), lambda b, ir: (b * _NR + ir, 0, 0)),
            pl.BlockSpec((1, 1, _N), lambda b, ir: (b, 0, 0)),
        ],
        out_shape=[
            jax.ShapeDtypeStruct((_B * _NR, 1, _RB), jnp.int32),
            jax.ShapeDtypeStruct((_B, 1, _N), jnp.float32),
        ],
        compiler_params=pltpu.CompilerParams(
            dimension_semantics=("arbitrary", "arbitrary")),
    )(gt)


def _att0_body(att_ref, ca_ref):
    A = att_ref[0]
    col_iota = lax.broadcasted_iota(jnp.int32, (_N, 1), 0).astype(jnp.float32)
    ca_ref[0, 0, :] = jnp.dot(A, col_iota,
                              preferred_element_type=jnp.float32)[:, 0].astype(jnp.int32)


def _att0_pass(att):
    nb = _SPLIT // _RB
    return pl.pallas_call(
        _att0_body,
        grid=(nb,),
        in_specs=[pl.BlockSpec((1, _RB, _N), lambda i: (0, i, 0))],
        out_specs=[pl.BlockSpec((1, 1, _RB), lambda i: (i, 0, 0))],
        out_shape=[jax.ShapeDtypeStruct((nb, 1, _RB), jnp.int32)],
        compiler_params=pltpu.CompilerParams(
            dimension_semantics=("arbitrary",)),
    )(att)[0]


def _chase_body(cg_hbm, ca0_hbm, casc_hbm, cgt_hbm, src_hbm, sc_out, fl_out,
                ca_v, cgt_v, cg_v, src_v, scv, flv):
    wid = lax.axis_index("s") * 2 + lax.axis_index("c")
    base = wid * _CHUNK
    b = base // _N
    ibase = base - b * _N

    @pl.when(base < _SPLIT)
    def _head():
        pltpu.sync_copy(ca0_hbm.at[pl.ds(base, _CHUNK)], ca_v)

    @pl.when(base >= _SPLIT)
    def _tail():
        off = pl.multiple_of(jnp.maximum(base - _SPLIT, 0), 8)
        pltpu.sync_copy(casc_hbm.at[pl.ds(off, _CHUNK)], ca_v)

    pltpu.sync_copy(cgt_hbm.at[pl.ds(b * _N, _N)], cgt_v)
    pltpu.sync_copy(cg_hbm.at[pl.ds(b * _N, _N)], cg_v)
    pltpu.sync_copy(src_hbm, src_v)
    s_vec = plsc.load_gather(src_v, [jnp.full((16,), b, jnp.int32)])
    lanes = lax.iota(jnp.int32, 16)
    for g in range(_GROUPS):
        i16 = ibase + g * 16 + lanes
        a16 = ca_v[pl.ds(g * 16, 16)]
        m16 = plsc.load_gather(cgt_v, [a16]).astype(jnp.int32)
        sc16 = plsc.load_gather(cg_v, [m16])
        flag = (i16 < s_vec) & (m16 != i16)
        scv[pl.ds(g * 16, 16)] = sc16
        flv[pl.ds(g * 16, 16)] = flag.astype(jnp.int32)
    pltpu.sync_copy(scv, sc_out.at[pl.ds(base, _CHUNK)])
    pltpu.sync_copy(flv, fl_out.at[pl.ds(base, _CHUNK)])


def _make_chase_kernel():
    return functools.partial(
        pl.kernel,
        mesh=plsc.VectorSubcoreMesh(core_axis_name="c", subcore_axis_name="s"),
        out_type=[
            jax.ShapeDtypeStruct((_B * _N,), jnp.int32),
            jax.ShapeDtypeStruct((_B * _N,), jnp.int32),
        ],
        compiler_params=pltpu.CompilerParams(needs_layout_passes=False),
        scratch_types=[
            pltpu.VMEM((_CHUNK,), jnp.int32),
            pltpu.VMEM((_N,), jnp.float32),
            pltpu.VMEM((_N,), jnp.int32),
            pltpu.VMEM((16,), jnp.int32),
            pltpu.VMEM((_CHUNK,), jnp.int32),
            pltpu.VMEM((_CHUNK,), jnp.int32),
        ],
    )(_chase_body)


def _pred_body(src_ref, tgt_ref, pred_ref, cg_ref, sc_ref, fl_ref,
               out_ref, acc_ref):
    b = pl.program_id(0)
    ir = pl.program_id(1)
    s = src_ref[b]
    t = tgt_ref[b]
    P = pred_ref[0]
    rows = lax.broadcasted_iota(jnp.int32, (_RB, _N), 0) + ir * _RB
    cols = lax.broadcasted_iota(jnp.int32, (_RB, _N), 1)
    rv = rows < s
    region = rv & (cols < t)
    l1mp = jnp.maximum(jnp.log(1.0 - P), -100.0)
    cgv = cg_ref[0, 0, :][:, None]
    scv = sc_ref[0, 0, :][:, None]
    ones_n = jnp.ones((_N, 1), jnp.float32)
    terms = jnp.where(region, l1mp, 0.0)
    s1 = -jnp.sum(jnp.dot(terms, ones_n, preferred_element_type=jnp.float32))
    p1 = jnp.dot(jnp.where(cols == cgv, P, 0.0), ones_n,
                 preferred_element_type=jnp.float32)[:, 0]
    p2 = jnp.dot(jnp.where(cols == scv, P, 0.0), ones_n,
                 preferred_element_type=jnp.float32)[:, 0]
    lp1 = jnp.maximum(jnp.log(p1), -100.0)
    l1mp1 = jnp.maximum(jnp.log(1.0 - p1), -100.0)
    corr = jnp.sum(jnp.where(rv[:, 0], l1mp1 - lp1, 0.0))
    flg = fl_ref[0, 0, :] != 0
    reg = jnp.sum(jnp.where(flg, p1 - p2, 0.0))
    part = s1 + corr - _REG_RATIO * reg

    @pl.when((b == 0) & (ir == 0))
    def _first():
        acc_ref[0] = part

    @pl.when((b != 0) | (ir != 0))
    def _rest():
        acc_ref[0] = acc_ref[0] + part

    @pl.when((b == _B - 1) & (ir == _NR - 1))
    def _flush():
        nsum = (src_ref[0] + src_ref[1] + src_ref[2] + src_ref[3]).astype(jnp.float32)
        out_ref[0] = acc_ref[0] / nsum


def _pred_pass(pred, cg, sc, fl, src_i, tgt_i):
    rowspec = pl.BlockSpec((1, 1, _RB), lambda b, ir: (b * _NR + ir, 0, 0))
    return pl.pallas_call(
        _pred_body,
        grid=(_B, _NR),
        in_specs=[
            pl.BlockSpec(memory_space=pltpu.SMEM),
            pl.BlockSpec(memory_space=pltpu.SMEM),
            pl.BlockSpec((1, _RB, _N), lambda b, ir: (b, ir, 0)),
            rowspec,
            rowspec,
            rowspec,
        ],
        out_specs=[pl.BlockSpec(memory_space=pltpu.SMEM)],
        out_shape=[jax.ShapeDtypeStruct((1,), jnp.float32)],
        scratch_shapes=[pltpu.SMEM((1,), jnp.float32)],
        compiler_params=pltpu.CompilerParams(
            dimension_semantics=("arbitrary", "arbitrary")),
    )(src_i, tgt_i, pred, cg, sc, fl)


def kernel(pred_dsmat, pred_perm, pred_perm_att, gt_perm, src_ns, tgt_ns):
    pred = pred_dsmat.astype(jnp.float32)
    gt = gt_perm.astype(jnp.float32)
    att = pred_perm_att.astype(jnp.float32)
    src_i = src_ns.astype(jnp.int32)
    tgt_i = tgt_ns.astype(jnp.int32)
    ca_sc = _make_att_kernel()(att.reshape(_B * _N, _N))
    cg, cgt = _gt_pass(gt)
    ca0 = _att0_pass(att)
    src_pad = jnp.zeros((16,), jnp.int32).at[:_B].set(src_i)
    sc, fl = _make_chase_kernel()(
        cg.reshape(_B * _N),
        ca0.reshape(_SPLIT),
        ca_sc,
        cgt.reshape(_B * _N),
        src_pad,
    )
    out = _pred_pass(
        pred,
        cg,
        sc.reshape(_B * _NR, 1, _RB),
        fl.reshape(_B * _NR, 1, _RB),
        src_i,
        tgt_i,
    )[0]
    return out[0]


# speedup vs baseline: 1.0251x; 1.0017x over previous
import functools

import jax
import jax.numpy as jnp
from jax import lax
from jax.experimental import pallas as pl
from jax.experimental.pallas import tpu as pltpu
from jax.experimental.pallas import tpu_sc as plsc

_B = 4
_N = 2048
_REG_RATIO = 0.1
_RB = 1024
_NR = _N // _RB
_NW = 32
_CHUNK = _B * _N // _NW
_GROUPS = _CHUNK // 16
_SPLIT = 2048
_SC_ROWS = _B * _N - _SPLIT
_APW = _SC_ROWS // _NW
_AGR = _APW // 16
_RCH = 16
_NCH = _APW // _RCH


def _att_body(att_hbm, out_hbm, va0, va1, ca_v, sem0, sem1):
    wid = lax.axis_index("s") * 2 + lax.axis_index("c")
    base = _SPLIT + wid * _APW
    lanes = lax.iota(jnp.int32, 16)
    lanesf = lanes.astype(jnp.float32)
    pltpu.async_copy(att_hbm.at[pl.ds(base, _RCH)], va0, sem0)

    def row_sum(va, k):
        def jbody(j, accs):
            a0, a1, a2, a3 = accs
            o = j * 128
            jf = o.astype(jnp.float32) + lanesf
            a0 = a0 + va[k, pl.ds(o, 16)] * jf
            a1 = a1 + va[k, pl.ds(o + 16, 16)] * (jf + 16.0)
            a2 = a2 + va[k, pl.ds(o + 32, 16)] * (jf + 32.0)
            a3 = a3 + va[k, pl.ds(o + 48, 16)] * (jf + 48.0)
            a0 = a0 + va[k, pl.ds(o + 64, 16)] * (jf + 64.0)
            a1 = a1 + va[k, pl.ds(o + 80, 16)] * (jf + 80.0)
            a2 = a2 + va[k, pl.ds(o + 96, 16)] * (jf + 96.0)
            a3 = a3 + va[k, pl.ds(o + 112, 16)] * (jf + 112.0)
            return (a0, a1, a2, a3)
        z = jnp.zeros((16,), jnp.float32)
        a0, a1, a2, a3 = lax.fori_loop(0, _N // 128, jbody, (z, z, z, z))
        return jnp.sum((a0 + a1) + (a2 + a3))

    def chunk_body(c, ca_vec):
        even = (c & 1) == 0
        nxt = c + 1

        @pl.when(even)
        def _w0():
            pltpu.make_async_copy(att_hbm.at[pl.ds(0, _RCH)], va0, sem0).wait()

        @pl.when(~even)
        def _w1():
            pltpu.make_async_copy(att_hbm.at[pl.ds(0, _RCH)], va1, sem1).wait()

        @pl.when(nxt < _NCH)
        def _issue():
            off = base + nxt * _RCH

            @pl.when(even)
            def _i1():
                pltpu.async_copy(att_hbm.at[pl.ds(off, _RCH)], va1, sem1)

            @pl.when(~even)
            def _i0():
                pltpu.async_copy(att_hbm.at[pl.ds(off, _RCH)], va0, sem0)

        ca_vec = jnp.zeros((16,), jnp.float32)
        for k in range(_RCH):
            s0 = lax.cond(even,
                          lambda: row_sum(va0, k),
                          lambda: row_sum(va1, k))
            ca_vec = jnp.where(lanes == k, s0, ca_vec)

        ca_v[pl.ds(c * 16, 16)] = ca_vec.astype(jnp.int32)
        return 0

    lax.fori_loop(0, _NCH, chunk_body, 0)
    pltpu.sync_copy(ca_v, out_hbm.at[pl.ds(wid * _APW, _APW)])


def _make_att_kernel():
    return functools.partial(
        pl.kernel,
        mesh=plsc.VectorSubcoreMesh(core_axis_name="c", subcore_axis_name="s"),
        out_type=jax.ShapeDtypeStruct((_SC_ROWS,), jnp.int32),
        compiler_params=pltpu.CompilerParams(needs_layout_passes=False),
        scratch_types=[
            pltpu.VMEM((_RCH, _N), jnp.float32),
            pltpu.VMEM((_RCH, _N), jnp.float32),
            pltpu.VMEM((_APW,), jnp.int32),
            pltpu.SemaphoreType.DMA,
            pltpu.SemaphoreType.DMA,
        ],
    )(_att_body)


def _gt_body(gt_ref, cg_ref, cgt_ref):
    ir = pl.program_id(1)
    G = gt_ref[0]
    col_iota = lax.broadcasted_iota(jnp.int32, (_N, 1), 0).astype(jnp.float32)
    row_iota = (lax.broadcasted_iota(jnp.int32, (1, _RB), 1) + ir * _RB).astype(jnp.float32)
    cg_ref[0, 0, :] = jnp.dot(G, col_iota,
                              preferred_element_type=jnp.float32)[:, 0].astype(jnp.int32)
    cgt_part = jnp.dot(row_iota, G, preferred_element_type=jnp.float32)[0]

    @pl.when(ir == 0)
    def _init():
        cgt_ref[0, 0, :] = cgt_part

    @pl.when(ir != 0)
    def _acc():
        cgt_ref[0, 0, :] = cgt_ref[0, 0, :] + cgt_part


def _gt_pass(gt):
    return pl.pallas_call(
        _gt_body,
        grid=(_B, _NR),
        in_specs=[pl.BlockSpec((1, _RB, _N), lambda b, ir: (b, ir, 0))],
        out_specs=[
            pl.BlockSpec((1, 1, _RB), lambda b, ir: (b * _NR + ir, 0, 0)),
            pl.BlockSpec((1, 1, _N), lambda b, ir: (b, 0, 0)),
        ],
        out_shape=[
            jax.ShapeDtypeStruct((_B * _NR, 1, _RB), jnp.int32),
            jax.ShapeDtypeStruct((_B, 1, _N), jnp.float32),
        ],
        compiler_params=pltpu.CompilerParams(
            dimension_semantics=("arbitrary", "arbitrary")),
    )(gt)


def _att0_body(att_ref, ca_ref):
    A = att_ref[0]
    col_iota = lax.broadcasted_iota(jnp.int32, (_N, 1), 0).astype(jnp.float32)
    ca_ref[0, 0, :] = jnp.dot(A, col_iota,
                              preferred_element_type=jnp.float32)[:, 0].astype(jnp.int32)


def _att0_pass(att):
    nb = _SPLIT // _RB
    return pl.pallas_call(
        _att0_body,
        grid=(nb,),
        in_specs=[pl.BlockSpec((1, _RB, _N), lambda i: (0, i, 0))],
        out_specs=[pl.BlockSpec((1, 1, _RB), lambda i: (i, 0, 0))],
        out_shape=[jax.ShapeDtypeStruct((nb, 1, _RB), jnp.int32)],
        compiler_params=pltpu.CompilerParams(
            dimension_semantics=("arbitrary",)),
    )(att)[0]


def _chase_body(cg_hbm, ca0_hbm, casc_hbm, cgt_hbm, src_hbm, sc_out, fl_out,
                ca_v, cgt_v, cg_v, src_v, scv, flv):
    wid = lax.axis_index("s") * 2 + lax.axis_index("c")
    base = wid * _CHUNK
    b = base // _N
    ibase = base - b * _N

    @pl.when(base < _SPLIT)
    def _head():
        pltpu.sync_copy(ca0_hbm.at[pl.ds(base, _CHUNK)], ca_v)

    @pl.when(base >= _SPLIT)
    def _tail():
        off = pl.multiple_of(jnp.maximum(base - _SPLIT, 0), 8)
        pltpu.sync_copy(casc_hbm.at[pl.ds(off, _CHUNK)], ca_v)

    pltpu.sync_copy(cgt_hbm.at[pl.ds(b * _N, _N)], cgt_v)
    pltpu.sync_copy(cg_hbm.at[pl.ds(b * _N, _N)], cg_v)
    pltpu.sync_copy(src_hbm, src_v)
    s_vec = plsc.load_gather(src_v, [jnp.full((16,), b, jnp.int32)])
    lanes = lax.iota(jnp.int32, 16)
    for g in range(_GROUPS):
        i16 = ibase + g * 16 + lanes
        a16 = ca_v[pl.ds(g * 16, 16)]
        m16 = plsc.load_gather(cgt_v, [a16]).astype(jnp.int32)
        sc16 = plsc.load_gather(cg_v, [m16])
        flag = (i16 < s_vec) & (m16 != i16)
        scv[pl.ds(g * 16, 16)] = sc16
        flv[pl.ds(g * 16, 16)] = flag.astype(jnp.int32)
    pltpu.sync_copy(scv, sc_out.at[pl.ds(base, _CHUNK)])
    pltpu.sync_copy(flv, fl_out.at[pl.ds(base, _CHUNK)])


def _make_chase_kernel():
    return functools.partial(
        pl.kernel,
        mesh=plsc.VectorSubcoreMesh(core_axis_name="c", subcore_axis_name="s"),
        out_type=[
            jax.ShapeDtypeStruct((_B * _N,), jnp.int32),
            jax.ShapeDtypeStruct((_B * _N,), jnp.int32),
        ],
        compiler_params=pltpu.CompilerParams(needs_layout_passes=False),
        scratch_types=[
            pltpu.VMEM((_CHUNK,), jnp.int32),
            pltpu.VMEM((_N,), jnp.float32),
            pltpu.VMEM((_N,), jnp.int32),
            pltpu.VMEM((16,), jnp.int32),
            pltpu.VMEM((_CHUNK,), jnp.int32),
            pltpu.VMEM((_CHUNK,), jnp.int32),
        ],
    )(_chase_body)


def _pred_body(src_ref, tgt_ref, pred_ref, cg_ref, sc_ref, fl_ref,
               out_ref, acc_ref):
    b = pl.program_id(0)
    ir = pl.program_id(1)
    s = src_ref[b]
    t = tgt_ref[b]
    P = pred_ref[0]
    rows = lax.broadcasted_iota(jnp.int32, (_RB, _N), 0) + ir * _RB
    cols = lax.broadcasted_iota(jnp.int32, (_RB, _N), 1)
    rv = rows < s
    region = rv & (cols < t)
    l1mp = jnp.maximum(jnp.log(1.0 - P), -100.0)
    cgv = cg_ref[0, 0, :][:, None]
    scv = sc_ref[0, 0, :][:, None]
    ones_n = jnp.ones((_N, 1), jnp.float32)
    terms = jnp.where(region, l1mp, 0.0)
    s1 = -jnp.sum(jnp.dot(terms, ones_n, preferred_element_type=jnp.float32))
    p1 = jnp.dot(jnp.where(cols == cgv, P, 0.0), ones_n,
                 preferred_element_type=jnp.float32)[:, 0]
    p2 = jnp.dot(jnp.where(cols == scv, P, 0.0), ones_n,
                 preferred_element_type=jnp.float32)[:, 0]
    lp1 = jnp.maximum(jnp.log(p1), -100.0)
    l1mp1 = jnp.maximum(jnp.log(1.0 - p1), -100.0)
    corr = jnp.sum(jnp.where(rv[:, 0], l1mp1 - lp1, 0.0))
    flg = fl_ref[0, 0, :] != 0
    reg = jnp.sum(jnp.where(flg, p1 - p2, 0.0))
    part = s1 + corr - _REG_RATIO * reg

    @pl.when((b == 0) & (ir == 0))
    def _first():
        acc_ref[0] = part

    @pl.when((b != 0) | (ir != 0))
    def _rest():
        acc_ref[0] = acc_ref[0] + part

    @pl.when((b == _B - 1) & (ir == _NR - 1))
    def _flush():
        nsum = (src_ref[0] + src_ref[1] + src_ref[2] + src_ref[3]).astype(jnp.float32)
        out_ref[0] = acc_ref[0] / nsum


def _pred_pass(pred, cg, sc, fl, src_i, tgt_i):
    rowspec = pl.BlockSpec((1, 1, _RB), lambda b, ir: (b * _NR + ir, 0, 0))
    return pl.pallas_call(
        _pred_body,
        grid=(_B, _NR),
        in_specs=[
            pl.BlockSpec(memory_space=pltpu.SMEM),
            pl.BlockSpec(memory_space=pltpu.SMEM),
            pl.BlockSpec((1, _RB, _N), lambda b, ir: (b, ir, 0)),
            rowspec,
            rowspec,
            rowspec,
        ],
        out_specs=[pl.BlockSpec(memory_space=pltpu.SMEM)],
        out_shape=[jax.ShapeDtypeStruct((1,), jnp.float32)],
        scratch_shapes=[pltpu.SMEM((1,), jnp.float32)],
        compiler_params=pltpu.CompilerParams(
            dimension_semantics=("arbitrary", "arbitrary")),
    )(src_i, tgt_i, pred, cg, sc, fl)


def kernel(pred_dsmat, pred_perm, pred_perm_att, gt_perm, src_ns, tgt_ns):
    pred = pred_dsmat.astype(jnp.float32)
    gt = gt_perm.astype(jnp.float32)
    att = pred_perm_att.astype(jnp.float32)
    src_i = src_ns.astype(jnp.int32)
    tgt_i = tgt_ns.astype(jnp.int32)
    ca_sc = _make_att_kernel()(att.reshape(_B * _N, _N))
    cg, cgt = _gt_pass(gt)
    ca0 = _att0_pass(att)
    src_pad = jnp.zeros((16,), jnp.int32).at[:_B].set(src_i)
    sc, fl = _make_chase_kernel()(
        cg.reshape(_B * _N),
        ca0.reshape(_SPLIT),
        ca_sc,
        cgt.reshape(_B * _N),
        src_pad,
    )
    out = _pred_pass(
        pred,
        cg,
        sc.reshape(_B * _NR, 1, _RB),
        fl.reshape(_B * _NR, 1, _RB),
        src_i,
        tgt_i,
    )[0]
    return out[0]
